# Initial kernel scaffold; baseline (speedup 1.0000x reference)
import jax
import jax.numpy as jnp
from jax.experimental import pallas as pl


def kernel(refined_scores, original_scores, node_labels, batch):
    raise NotImplementedError("write your pallas kernel here")


# speedup vs baseline: 2.6334x; 2.6334x over previous
import functools

import jax
import jax.numpy as jnp
from jax import lax
from jax.experimental import pallas as pl
from jax.experimental.pallas import tpu as pltpu
from jax.experimental.pallas import tpu_sc as plsc

_N = 10000
_G = 8
_NC = 2
_NS = 16
_NW = _NC * _NS
_CH = 320
_NBLK = _N // 16
_NPAD = _N + 16
_MARGIN = 0.1
_NEG = -1e30


def _sc_body(r_hbm, o_hbm, lab_hbm, bat_hbm, part_hbm,
             r_v, o_v, bs_v, lab_v, bat_v, bnd_v, out_v):
    wid = lax.axis_index("s") * _NC + lax.axis_index("c")
    iot = lax.iota(jnp.int32, 16)

    pltpu.sync_copy(r_hbm, r_v.at[pl.ds(0, _N)])
    pltpu.sync_copy(o_hbm, o_v.at[pl.ds(0, _N)])
    pltpu.sync_copy(lab_hbm, lab_v.at[pl.ds(0, _N)])
    pltpu.sync_copy(bat_hbm, bat_v.at[pl.ds(0, _N)])

    def bs_step(k, _):
        sl = pl.ds(k * 16, 16)
        bs_v[sl] = jnp.where(lab_v[sl] == 0, r_v[sl], _NEG)
        return 0
    lax.fori_loop(0, _NBLK, bs_step, 0)

    bnd = jnp.where(iot == 8, _N, 0).astype(jnp.int32)
    for g in range(1, _G):
        def bstep(_, lohi, g=g):
            lo, hi = lohi
            mid = (lo + hi) // 2
            pred = bat_v[pl.ds(mid, 16)][0] < g
            return (jnp.where(pred, mid + 1, lo), jnp.where(pred, hi, mid))
        lo, _hi = lax.fori_loop(0, 14, bstep, (jnp.int32(0), jnp.int32(_N)))
        bnd = bnd + jnp.where(iot == g, lo, 0)
    bnd_v[pl.ds(0, 16)] = bnd
    bnd_v[pl.ds(16, 16)] = jnp.zeros((16,), jnp.int32)

    base_i = wid * _CH
    nrows = jnp.minimum(_CH, _N - base_i)

    zero16 = jnp.zeros((16,), jnp.float32)

    def row_step(il, carry):
        num_accs, gold_acc = carry
        i = base_i + il
        gold = lab_v[pl.ds(i, 16)][0] > 0
        g = bat_v[pl.ds(i, 16)][0]
        st = bnd_v[pl.ds(g, 16)][0]
        en = bnd_v[pl.ds(g + 1, 16)][0]
        c0 = _MARGIN - r_v[pl.ds(i, 16)][0]
        jb0 = st // 16
        jb1 = jnp.where(gold, (en + 15) // 16, jb0)

        def jstep(jb, acc):
            base = jb * 16
            jv = base + iot
            v = jnp.maximum(c0 + bs_v[pl.ds(base, 16)], 0.0)
            v = jnp.where(jv >= st, v, 0.0)
            v = jnp.where(jv < en, v, 0.0)
            return acc + v

        racc = lax.fori_loop(jb0, jb1, jstep, zero16)
        goldf = jnp.where(gold, jnp.float32(1.0), jnp.float32(0.0))
        num_accs = tuple(
            num_accs[gg]
            + racc * jnp.where(g == gg, jnp.float32(1.0), jnp.float32(0.0))
            for gg in range(_G))
        gold_acc = gold_acc + jnp.where(iot == g, goldf, 0.0)
        return (num_accs, gold_acc)

    num_accs, gold_acc = lax.fori_loop(
        0, nrows, row_step, ((zero16,) * _G, zero16))

    kb0 = wid * (_CH // 16)
    kb1 = jnp.minimum(kb0 + (_CH // 16), _NBLK)

    def ar_step(kb, c):
        a_acc, r_acc = c
        sl = pl.ds(kb * 16, 16)
        d = r_v[sl] - o_v[sl]
        return (a_acc + d * d, r_acc + jnp.abs(d))

    a_acc, r_acc = lax.fori_loop(kb0, kb1, ar_step, (zero16, zero16))

    for gg in range(_G):
        out_v[pl.ds(gg * 16, 16)] = num_accs[gg]
    out_v[pl.ds(128, 16)] = gold_acc
    out_v[pl.ds(144, 16)] = a_acc
    out_v[pl.ds(160, 16)] = r_acc
    out_v[pl.ds(176, 16)] = zero16
    pltpu.sync_copy(out_v, part_hbm.at[wid])


_sc_kernel = functools.partial(
    pl.kernel,
    out_type=jax.ShapeDtypeStruct((_NW, 192), jnp.float32),
    mesh=plsc.VectorSubcoreMesh(
        core_axis_name="c", subcore_axis_name="s",
        num_cores=_NC, num_subcores=_NS),
    scratch_types=[
        pltpu.VMEM((_NPAD,), jnp.float32),
        pltpu.VMEM((_NPAD,), jnp.float32),
        pltpu.VMEM((_NPAD,), jnp.float32),
        pltpu.VMEM((_NPAD,), jnp.int32),
        pltpu.VMEM((_NPAD,), jnp.int32),
        pltpu.VMEM((32,), jnp.int32),
        pltpu.VMEM((192,), jnp.float32),
    ],
)(_sc_body)


def _combine_body(part_ref, bat_ref, out_ref):
    p = part_ref[...]
    b = bat_ref[...]
    asum = jnp.sum(p[:, 144:160])
    rsum = jnp.sum(p[:, 160:176])
    rank_num = jnp.float32(0.0)
    rank_den = jnp.float32(0.0)
    for g in range(_G):
        num_g = jnp.sum(p[:, g * 16:(g + 1) * 16])
        gold_g = jnp.sum(p[:, 128 + g:129 + g])
        size_g = jnp.sum((b == g).astype(jnp.float32))
        cnt = gold_g * (size_g - gold_g)
        lg = jnp.where(cnt > 0, num_g / jnp.maximum(cnt, 1.0), 0.0)
        rank_num = rank_num + lg
        rank_den = rank_den + jnp.where(cnt > 0, 1.0, 0.0)
    rank = jnp.where(rank_den > 0, rank_num / jnp.maximum(rank_den, 1.0), 0.0)
    align = asum / _N
    reg = rsum / _N
    total = rank + 0.5 * align + 0.1 * reg
    ri = lax.broadcasted_iota(jnp.int32, (8, 128), 0)
    li = lax.broadcasted_iota(jnp.int32, (8, 128), 1)
    vals = (jnp.where(li == 0, total, 0.0) + jnp.where(li == 1, rank, 0.0)
            + jnp.where(li == 2, align, 0.0) + jnp.where(li == 3, reg, 0.0))
    out_ref[...] = jnp.where(ri == 0, vals, 0.0)


def _combine(part, bat_pad):
    return pl.pallas_call(
        _combine_body,
        out_shape=jax.ShapeDtypeStruct((8, 128), jnp.float32),
    )(part, bat_pad)


def kernel(refined_scores, original_scores, node_labels, batch):
    r = refined_scores.astype(jnp.float32)
    o = original_scores.astype(jnp.float32)
    lab = node_labels.astype(jnp.int32)
    bat = batch.astype(jnp.int32)
    part = _sc_kernel(r, o, lab, bat)
    bat_pad = jnp.pad(bat, (0, _NW * _CH - _N),
                      constant_values=_G).reshape(80, 128)
    res = _combine(part, bat_pad)
    return (res[0, 0], res[0, 1], res[0, 2], res[0, 3])


# speedup vs baseline: 5.3885x; 2.0462x over previous
import functools

import jax
import jax.numpy as jnp
from jax import lax
from jax.experimental import pallas as pl
from jax.experimental.pallas import tpu as pltpu
from jax.experimental.pallas import tpu_sc as plsc

_N = 10000
_G = 8
_NC = 2
_NS = 16
_NW = _NC * _NS
_CH = 320
_NBLK = _N // 16
_NPAD = _N + 16
_MARGIN = 0.1
_NEG = -1e30


def _tree_sum(vs):
    while len(vs) > 1:
        vs = [vs[i] + vs[i + 1] for i in range(0, len(vs) - 1, 2)] + (
            [vs[-1]] if len(vs) % 2 else [])
    return vs[0]


def _sc_body(r_hbm, o_hbm, lab_hbm, bat_hbm, part_hbm,
             r_v, o_v, bs_v, lab_v, bat_v, out_v):
    wid = lax.axis_index("s") * _NC + lax.axis_index("c")
    iot = lax.iota(jnp.int32, 16)
    zero16 = jnp.zeros((16,), jnp.float32)

    pltpu.sync_copy(r_hbm, r_v.at[pl.ds(0, _N)])
    pltpu.sync_copy(o_hbm, o_v.at[pl.ds(0, _N)])
    pltpu.sync_copy(lab_hbm, lab_v.at[pl.ds(0, _N)])
    pltpu.sync_copy(bat_hbm, bat_v.at[pl.ds(0, _N)])

    def bs_step(k, _):
        sl = pl.ds(k * 16, 16)
        bs_v[sl] = jnp.where(lab_v[sl] == 0, r_v[sl], _NEG)
        return 0
    lax.fori_loop(0, _NBLK, bs_step, 0)

    starts = [jnp.int32(0)]
    for g in range(1, _G):
        def bstep(_, lohi, g=g):
            lo, hi = lohi
            mid = (lo + hi) // 2
            pred = bat_v[pl.ds(mid, 16)][0] < g
            return (jnp.where(pred, mid + 1, lo), jnp.where(pred, hi, mid))
        lo, _hi = lax.fori_loop(0, 14, bstep, (jnp.int32(0), jnp.int32(_N)))
        starts.append(lo)
    starts.append(jnp.int32(_N))

    base_i = wid * _CH

    num_vecs = []
    gold_vecs = []
    for g in range(_G):
        st = starts[g]
        en = starts[g + 1]
        a0 = st // 16
        b0 = en // 16
        jvA = a0 * 16 + iot
        vA = bs_v[pl.ds(a0 * 16, 16)]
        vA = jnp.where(jvA >= st, vA, _NEG)
        vA = jnp.where(jvA < en, vA, _NEG)
        jvB = b0 * 16 + iot
        vB = bs_v[pl.ds(b0 * 16, 16)]
        vB = jnp.where(jvB >= (a0 + 1) * 16, vB, _NEG)
        vB = jnp.where(jvB < en, vB, _NEG)

        rlo = jnp.maximum(base_i, st)
        rhi = jnp.minimum(base_i + _CH, en)
        nrb = jnp.maximum(0, (rhi - rlo + 15) // 16)

        def rb_step(rb, c, rlo=rlo, rhi=rhi, a0=a0, b0=b0, vA=vA, vB=vB):
            nacc, gacc = c
            rbase = rlo + rb * 16
            rowjv = rbase + iot
            av = r_v[pl.ds(rbase, 16)]
            lv = lab_v[pl.ds(rbase, 16)]
            c0 = _MARGIN - av
            c0 = jnp.where(lv > 0, c0, _NEG)
            c0 = jnp.where(rowjv < rhi, c0, _NEG)
            c0b = [c0[k] + zero16 for k in range(16)]

            def jstep(jb, accs):
                b = bs_v[pl.ds(jb * 16, 16)]
                return tuple(accs[k] + jnp.maximum(b + c0b[k], 0.0)
                             for k in range(16))

            accs = lax.fori_loop(a0 + 1, b0, jstep, (zero16,) * 16)
            for ev in (vA, vB):
                accs = tuple(accs[k] + jnp.maximum(ev + c0b[k], 0.0)
                             for k in range(16))
            nacc = nacc + _tree_sum(list(accs))
            rmask = jnp.where(rowjv < rhi, jnp.float32(1.0), jnp.float32(0.0))
            gacc = gacc + jnp.where(lv > 0, rmask, 0.0)
            return (nacc, gacc)

        nv, gv = lax.fori_loop(0, nrb, rb_step, (zero16, zero16))
        num_vecs.append(nv)
        gold_vecs.append(gv)

    kb0 = wid * (_CH // 16)
    kb1 = jnp.minimum(kb0 + (_CH // 16), _NBLK)

    def ar_step(kb, c):
        a_acc, r_acc = c
        sl = pl.ds(kb * 16, 16)
        d = r_v[sl] - o_v[sl]
        return (a_acc + d * d, r_acc + jnp.abs(d))

    a_acc, r_acc = lax.fori_loop(kb0, kb1, ar_step, (zero16, zero16))

    for g in range(_G):
        out_v[pl.ds(g * 16, 16)] = num_vecs[g]
        out_v[pl.ds(128 + g * 16, 16)] = gold_vecs[g]
    out_v[pl.ds(256, 16)] = a_acc
    out_v[pl.ds(272, 16)] = r_acc
    pltpu.sync_copy(out_v, part_hbm.at[wid])


_sc_kernel = functools.partial(
    pl.kernel,
    out_type=jax.ShapeDtypeStruct((_NW, 288), jnp.float32),
    mesh=plsc.VectorSubcoreMesh(
        core_axis_name="c", subcore_axis_name="s",
        num_cores=_NC, num_subcores=_NS),
    scratch_types=[
        pltpu.VMEM((_NPAD,), jnp.float32),
        pltpu.VMEM((_NPAD,), jnp.float32),
        pltpu.VMEM((_NPAD,), jnp.float32),
        pltpu.VMEM((_NPAD,), jnp.int32),
        pltpu.VMEM((_NPAD,), jnp.int32),
        pltpu.VMEM((288,), jnp.float32),
    ],
)(_sc_body)


def _combine_body(part_ref, bat_ref, out_ref):
    p = part_ref[...]
    b = bat_ref[...]
    asum = jnp.sum(p[:, 256:272])
    rsum = jnp.sum(p[:, 272:288])
    rank_num = jnp.float32(0.0)
    rank_den = jnp.float32(0.0)
    for g in range(_G):
        num_g = jnp.sum(p[:, g * 16:(g + 1) * 16])
        gold_g = jnp.sum(p[:, 128 + g * 16:144 + g * 16])
        size_g = jnp.sum((b == g).astype(jnp.float32))
        cnt = gold_g * (size_g - gold_g)
        lg = jnp.where(cnt > 0, num_g / jnp.maximum(cnt, 1.0), 0.0)
        rank_num = rank_num + lg
        rank_den = rank_den + jnp.where(cnt > 0, 1.0, 0.0)
    rank = jnp.where(rank_den > 0, rank_num / jnp.maximum(rank_den, 1.0), 0.0)
    align = asum / _N
    reg = rsum / _N
    total = rank + 0.5 * align + 0.1 * reg
    ri = lax.broadcasted_iota(jnp.int32, (8, 128), 0)
    li = lax.broadcasted_iota(jnp.int32, (8, 128), 1)
    vals = (jnp.where(li == 0, total, 0.0) + jnp.where(li == 1, rank, 0.0)
            + jnp.where(li == 2, align, 0.0) + jnp.where(li == 3, reg, 0.0))
    out_ref[...] = jnp.where(ri == 0, vals, 0.0)


def _combine(part, bat_pad):
    return pl.pallas_call(
        _combine_body,
        out_shape=jax.ShapeDtypeStruct((8, 128), jnp.float32),
    )(part, bat_pad)


def kernel(refined_scores, original_scores, node_labels, batch):
    r = refined_scores.astype(jnp.float32)
    o = original_scores.astype(jnp.float32)
    lab = node_labels.astype(jnp.int32)
    bat = batch.astype(jnp.int32)
    part = _sc_kernel(r, o, lab, bat)
    bat_pad = jnp.pad(bat, (0, _NW * _CH - _N),
                      constant_values=_G).reshape(80, 128)
    res = _combine(part, bat_pad)
    return (res[0, 0], res[0, 1], res[0, 2], res[0, 3])
